# baseline (speedup 1.0000x reference)
import functools

import jax
import jax.numpy as jnp
from jax import lax
from jax.experimental import pallas as pl
from jax.experimental.pallas import tpu as pltpu
from jax.experimental.pallas import tpu_sc as plsc

_VOCAB = 100000
_NUM_CAT = 26
_EMB_DIM = 32
_LAMBDA = 0.001

_LANES = 16


def _sc_hist_body(v5_hbm, hist_hbm, idbuf, hist_v, sem0, sem1, *, n_t, n_reg):
    nbt = v5_hbm.shape[2]
    wid = lax.axis_index("s") * 2 + lax.axis_index("c")

    @pl.when(wid < _NUM_CAT)
    def _():
        @plsc.parallel_loop(0, _VOCAB // _LANES, unroll=16)
        def _(j):
            hist_v[pl.ds(j * _LANES, _LANES)] = jnp.zeros((_LANES,), jnp.int32)

        sems = (sem0, sem1)
        col = n_reg + wid
        et = col // 8
        r = col % 8

        def dma(t, b):
            return pltpu.make_async_copy(
                v5_hbm.at[t, et, :, r], idbuf.at[b], sems[b]
            )

        def process(b):
            @plsc.parallel_loop(0, nbt * 8, unroll=8)
            def _(i):
                v = idbuf[b, i >> 3, pl.ds((i & 7) * _LANES, _LANES)]
                vi = v.astype(jnp.int32)
                cnt, last = plsc.scan_count(vi)
                plsc.addupdate_scatter(hist_v, [vi], cnt, mask=last)

        dma(0, 0).start()

        def outer(k, carry):
            t0 = 2 * k
            dma(t0 + 1, 1).start()
            dma(t0, 0).wait()
            process(0)
            dma(jnp.minimum(t0 + 2, n_t - 1), 0).start()
            dma(t0 + 1, 1).wait()
            process(1)
            return carry

        lax.fori_loop(0, n_t // 2, outer, 0)
        dma(n_t - 1, 0).wait()

        pltpu.sync_copy(hist_v, hist_hbm.at[wid, 0])


def _sc_hist(v5, n_t, n_reg):
    mesh = plsc.VectorSubcoreMesh(
        core_axis_name="c", subcore_axis_name="s", num_cores=2, num_subcores=16
    )
    body = functools.partial(_sc_hist_body, n_t=n_t, n_reg=n_reg)
    nbt = v5.shape[2]
    return pl.kernel(
        body,
        out_type=jax.ShapeDtypeStruct((_NUM_CAT, 1, _VOCAB), jnp.int32),
        mesh=mesh,
        scratch_types=[
            pltpu.VMEM((2, nbt, 128), jnp.float32),
            pltpu.VMEM((_VOCAB,), jnp.int32),
            pltpu.SemaphoreType.DMA,
            pltpu.SemaphoreType.DMA,
        ],
        compiler_params=pltpu.CompilerParams(needs_layout_passes=False),
    )(v5)


_KR = 16
_KS = _EMB_DIM // _KR


def _tc_w2_body(tab_ref, out_ref):
    k = pl.program_id(1)
    x = tab_ref[...]
    p = jnp.sum(x * x, axis=0, keepdims=True)[None]

    @pl.when(k == 0)
    def _():
        out_ref[...] = p

    @pl.when(k > 0)
    def _():
        out_ref[...] += p


def _tc_w2(tables2):
    return pl.pallas_call(
        _tc_w2_body,
        grid=(_NUM_CAT, _KS),
        in_specs=[
            pl.BlockSpec((_KR, _VOCAB), lambda g, k: (g * _KS + k, 0)),
        ],
        out_specs=pl.BlockSpec((1, 1, _VOCAB), lambda g, k: (g, 0, 0)),
        out_shape=jax.ShapeDtypeStruct((_NUM_CAT, 1, _VOCAB), jnp.float32),
    )(tables2)


_CC = 13


def _tc_combine_body(w2_ref, hist_ref, out_ref, *, n_total):
    g = pl.program_id(0)

    @pl.when(g == 0)
    def _():
        out_ref[0, 0] = 0.0

    h = hist_ref[:, 0].astype(jnp.float32)
    a = lax.rsqrt(h / n_total + 1e-9)
    out_ref[0, 0] += jnp.sum(a * w2_ref[:, 0])


def _tc_combine(w2, hist, n_total):
    body = functools.partial(_tc_combine_body, n_total=float(n_total))
    return pl.pallas_call(
        body,
        grid=(_NUM_CAT // _CC,),
        in_specs=[
            pl.BlockSpec((_CC, 1, _VOCAB), lambda g: (g, 0, 0)),
            pl.BlockSpec((_CC, 1, _VOCAB), lambda g: (g, 0, 0)),
        ],
        out_specs=pl.BlockSpec(memory_space=pltpu.SMEM),
        out_shape=jax.ShapeDtypeStruct((1, 1), jnp.float32),
    )(w2, hist)


def kernel(inputs, tables):
    b, t, c = inputs.shape
    n_total = b * t
    inputs_t = lax.transpose(inputs, (1, 2, 0))
    tables2 = lax.transpose(tables, (0, 2, 1)).reshape(_NUM_CAT * _EMB_DIM, _VOCAB)
    v5 = lax.transpose(
        inputs_t.reshape(t, c // 8, 8, b // 128, 128), (0, 1, 3, 2, 4)
    )
    hist = _sc_hist(v5, t, c - _NUM_CAT)
    w2 = _tc_w2(tables2)
    total = _tc_combine(w2, hist, n_total)
    loss = (_LAMBDA / _VOCAB) * total[0, 0]
    return (inputs, loss)
